# initial kernel scaffold (unmeasured)
import jax
import jax.numpy as jnp
from jax import lax
from jax.experimental import pallas as pl
from jax.experimental.pallas import tpu as pltpu

N_DEV = 4


def kernel(x, Wg, Wu, Wd):
    m, k = x.shape
    d = Wd.shape[1]

    def body(x_ref, wg_ref, wu_ref, wd_ref, out_ref, comm_ref, send_sems, recv_sems):
        my_pos = lax.axis_index("i")
        left = (my_pos - 1) % N_DEV
        right = (my_pos + 1) % N_DEV

        barrier_sem = pltpu.get_barrier_semaphore()
        for nbr in [left, right]:
            pl.semaphore_signal(
                barrier_sem, inc=1,
                device_id=(nbr,), device_id_type=pl.DeviceIdType.MESH,
            )
        pl.semaphore_wait(barrier_sem, 2)

        gate = jnp.dot(x_ref[:, :], wg_ref[:, :], preferred_element_type=jnp.float32)
        up = jnp.dot(x_ref[:, :], wu_ref[:, :], preferred_element_type=jnp.float32)
        h = gate * (up * jax.nn.sigmoid(up))
        partial = jnp.dot(h, wd_ref[:, :], preferred_element_type=jnp.float32)

        out_ref[:, :] = partial
        comm_ref[0, :, :] = partial

        for hop in range(N_DEV - 1):
            send_slot = hop % 2
            recv_slot = (hop + 1) % 2
            rdma = pltpu.make_async_remote_copy(
                src_ref=comm_ref.at[send_slot],
                dst_ref=comm_ref.at[recv_slot],
                send_sem=send_sems.at[send_slot],
                recv_sem=recv_sems.at[recv_slot],
                device_id=(right,),
                device_id_type=pl.DeviceIdType.MESH,
            )
            rdma.start()
            rdma.wait()
            out_ref[:, :] += comm_ref[recv_slot, :, :]

    return pl.pallas_call(
        body,
        out_shape=jax.ShapeDtypeStruct((m, d), jnp.float32),
        in_specs=[pl.BlockSpec(memory_space=pltpu.VMEM)] * 4,
        out_specs=pl.BlockSpec(memory_space=pltpu.VMEM),
        scratch_shapes=[
            pltpu.VMEM((2, m, d), jnp.float32),
            pltpu.SemaphoreType.DMA((2,)),
            pltpu.SemaphoreType.DMA((2,)),
        ],
        compiler_params=pltpu.CompilerParams(collective_id=0),
    )(x, Wg, Wu, Wd)


# baseline (device time: 175234 ns/iter reference)
import jax
import jax.numpy as jnp
from jax import lax
from jax.experimental import pallas as pl
from jax.experimental.pallas import tpu as pltpu

N_DEV = 4


def kernel(x, Wg, Wu, Wd):
    m, k = x.shape
    d = Wd.shape[1]

    def body(x_ref, wg_ref, wu_ref, wd_ref, out_ref, comm_ref, send_sems, recv_sems):
        my_pos = lax.axis_index("i")
        left = (my_pos - 1) % N_DEV
        right = (my_pos + 1) % N_DEV

        barrier_sem = pltpu.get_barrier_semaphore()
        for nbr in [left, right]:
            pl.semaphore_signal(
                barrier_sem, inc=1,
                device_id=(nbr,), device_id_type=pl.DeviceIdType.MESH,
            )
        pl.semaphore_wait(barrier_sem, 2)

        gate = jnp.dot(x_ref[:, :], wg_ref[:, :], preferred_element_type=jnp.float32)
        up = jnp.dot(x_ref[:, :], wu_ref[:, :], preferred_element_type=jnp.float32)
        h = gate * (up * jax.nn.sigmoid(up))
        partial = jnp.dot(h, wd_ref[:, :], preferred_element_type=jnp.float32)

        out_ref[:, :] = partial
        comm_ref[0, :, :] = partial

        for hop in range(N_DEV - 1):
            send_slot = hop % 2
            recv_slot = (hop + 1) % 2
            rdma = pltpu.make_async_remote_copy(
                src_ref=comm_ref.at[send_slot],
                dst_ref=comm_ref.at[recv_slot],
                send_sem=send_sems.at[send_slot],
                recv_sem=recv_sems.at[recv_slot],
                device_id=(right,),
                device_id_type=pl.DeviceIdType.MESH,
            )
            rdma.start()
            rdma.wait()
            out_ref[:, :] += comm_ref[recv_slot, :, :]

    return pl.pallas_call(
        body,
        out_shape=jax.ShapeDtypeStruct((m, d), jnp.float32),
        in_specs=[pl.BlockSpec(memory_space=pltpu.VMEM)] * 4,
        out_specs=pl.BlockSpec(memory_space=pltpu.VMEM),
        scratch_shapes=[
            pltpu.VMEM((2, m, d), jnp.float32),
            pltpu.SemaphoreType.DMA((2,)),
            pltpu.SemaphoreType.DMA((2,)),
        ],
        compiler_params=pltpu.CompilerParams(
            collective_id=0, vmem_limit_bytes=100 * 1024 * 1024
        ),
    )(x, Wg, Wu, Wd)


# device time: 67589 ns/iter; 2.5926x vs baseline; 2.5926x over previous
import jax
import jax.numpy as jnp
from jax import lax
from jax.experimental import pallas as pl
from jax.experimental.pallas import tpu as pltpu

N_DEV = 4


def kernel(x, Wg, Wu, Wd):
    m, k = x.shape
    d = Wd.shape[1]
    mc = m // N_DEV
    dh = d // 2

    def body(x_ref, wg_ref, wu_ref, wd_ref, out_ref,
             commR, commL, sendR, recvR, sendL, recvL):
        my = lax.axis_index("i")
        left = (my - 1) % N_DEV
        right = (my + 1) % N_DEV

        barrier_sem = pltpu.get_barrier_semaphore()
        for nbr in [left, right]:
            pl.semaphore_signal(
                barrier_sem, inc=1,
                device_id=(nbr,), device_id_type=pl.DeviceIdType.MESH,
            )
        pl.semaphore_wait(barrier_sem, 2)

        def compute_chunk(c):
            rows = pl.ds(c * mc, mc)
            xg = x_ref[rows, :]
            gate = jnp.dot(xg, wg_ref[:, :], preferred_element_type=jnp.float32)
            up = jnp.dot(xg, wu_ref[:, :], preferred_element_type=jnp.float32)
            h = gate * (up * jax.nn.sigmoid(up))
            return jnp.dot(h, wd_ref[:, :], preferred_element_type=jnp.float32)

        def hop(h):
            s_slot = h % 2
            r_slot = (h + 1) % 2
            rR = pltpu.make_async_remote_copy(
                src_ref=commR.at[s_slot], dst_ref=commR.at[r_slot],
                send_sem=sendR.at[s_slot], recv_sem=recvR.at[r_slot],
                device_id=(right,), device_id_type=pl.DeviceIdType.MESH,
            )
            rL = pltpu.make_async_remote_copy(
                src_ref=commL.at[s_slot], dst_ref=commL.at[r_slot],
                send_sem=sendL.at[s_slot], recv_sem=recvL.at[r_slot],
                device_id=(left,), device_id_type=pl.DeviceIdType.MESH,
            )
            rR.start()
            rL.start()
            return rR, rL

        p_own = compute_chunk(my)
        commR[0, :, :] = p_own[:, :dh]
        commL[0, :, :] = p_own[:, dh:]
        rR, rL = hop(0)

        p_m1 = compute_chunk((my - 1) % N_DEV)
        p_p1 = compute_chunk((my + 1) % N_DEV)
        rR.wait()
        rL.wait()
        commR[1, :, :] += p_m1[:, :dh]
        commL[1, :, :] += p_p1[:, dh:]
        rR, rL = hop(1)

        p_2 = compute_chunk((my + 2) % N_DEV)
        rR.wait()
        rL.wait()
        commR[0, :, :] += p_2[:, :dh]
        commL[0, :, :] += p_2[:, dh:]
        rR, rL = hop(2)

        rR.wait()
        rL.wait()
        commR[1, :, :] += p_p1[:, :dh]
        commL[1, :, :] += p_m1[:, dh:]
        out_ref[pl.ds(((my + 1) % N_DEV) * mc, mc), 0:dh] = commR[1, :, :]
        out_ref[pl.ds(((my - 1) % N_DEV) * mc, mc), dh:d] = commL[1, :, :]

        for h in range(3, 6):
            rR, rL = hop(h)
            rR.wait()
            rL.wait()
            r_slot = (h + 1) % 2
            cR = (my - (h - 3)) % N_DEV
            cL = (my + (h - 3)) % N_DEV
            out_ref[pl.ds(cR * mc, mc), 0:dh] = commR[r_slot, :, :]
            out_ref[pl.ds(cL * mc, mc), dh:d] = commL[r_slot, :, :]

    return pl.pallas_call(
        body,
        out_shape=jax.ShapeDtypeStruct((m, d), jnp.float32),
        in_specs=[pl.BlockSpec(memory_space=pltpu.VMEM)] * 4,
        out_specs=pl.BlockSpec(memory_space=pltpu.VMEM),
        scratch_shapes=[
            pltpu.VMEM((2, mc, dh), jnp.float32),
            pltpu.VMEM((2, mc, dh), jnp.float32),
            pltpu.SemaphoreType.DMA((2,)),
            pltpu.SemaphoreType.DMA((2,)),
            pltpu.SemaphoreType.DMA((2,)),
            pltpu.SemaphoreType.DMA((2,)),
        ],
        compiler_params=pltpu.CompilerParams(
            collective_id=0, vmem_limit_bytes=100 * 1024 * 1024
        ),
    )(x, Wg, Wu, Wd)


# device time: 53181 ns/iter; 3.2950x vs baseline; 1.2709x over previous
import jax
import jax.numpy as jnp
from jax import lax
from jax.experimental import pallas as pl
from jax.experimental.pallas import tpu as pltpu

N_DEV = 4


def kernel(x, Wg, Wu, Wd):
    m, k = x.shape
    d = Wd.shape[1]
    mc = m // N_DEV
    dh = d // 2

    def body(x_ref, wg_ref, wu_ref, wd_ref, out_ref,
             commR, commL, sendR, recvR, sendL, recvL):
        my = lax.axis_index("i")
        left = (my - 1) % N_DEV
        right = (my + 1) % N_DEV

        barrier_sem = pltpu.get_barrier_semaphore()
        for nbr in [left, right]:
            pl.semaphore_signal(
                barrier_sem, inc=1,
                device_id=(nbr,), device_id_type=pl.DeviceIdType.MESH,
            )
        pl.semaphore_wait(barrier_sem, 2)

        wg16 = wg_ref[:, :].astype(jnp.bfloat16)
        wu16 = wu_ref[:, :].astype(jnp.bfloat16)
        wd16 = wd_ref[:, :].astype(jnp.bfloat16)

        def compute_chunk(c):
            rows = pl.ds(c * mc, mc)
            xg = x_ref[rows, :].astype(jnp.bfloat16)
            gate = jnp.dot(xg, wg16, preferred_element_type=jnp.float32)
            up = jnp.dot(xg, wu16, preferred_element_type=jnp.float32)
            h = (gate * (up * jax.nn.sigmoid(up))).astype(jnp.bfloat16)
            return jnp.dot(h, wd16, preferred_element_type=jnp.float32).astype(
                jnp.bfloat16
            )

        def hop(h):
            s_slot = h % 2
            r_slot = (h + 1) % 2
            rR = pltpu.make_async_remote_copy(
                src_ref=commR.at[s_slot], dst_ref=commR.at[r_slot],
                send_sem=sendR.at[s_slot], recv_sem=recvR.at[r_slot],
                device_id=(right,), device_id_type=pl.DeviceIdType.MESH,
            )
            rL = pltpu.make_async_remote_copy(
                src_ref=commL.at[s_slot], dst_ref=commL.at[r_slot],
                send_sem=sendL.at[s_slot], recv_sem=recvL.at[r_slot],
                device_id=(left,), device_id_type=pl.DeviceIdType.MESH,
            )
            rR.start()
            rL.start()
            return rR, rL

        p_own = compute_chunk(my)
        commR[0, :, :] = p_own[:, :dh]
        commL[0, :, :] = p_own[:, dh:]
        rR, rL = hop(0)

        p_m1 = compute_chunk((my - 1) % N_DEV)
        p_p1 = compute_chunk((my + 1) % N_DEV)
        rR.wait()
        rL.wait()
        commR[1, :, :] += p_m1[:, :dh]
        commL[1, :, :] += p_p1[:, dh:]
        rR, rL = hop(1)

        p_2 = compute_chunk((my + 2) % N_DEV)
        rR.wait()
        rL.wait()
        commR[0, :, :] += p_2[:, :dh]
        commL[0, :, :] += p_2[:, dh:]
        rR, rL = hop(2)

        rR.wait()
        rL.wait()
        commR[1, :, :] += p_p1[:, :dh]
        commL[1, :, :] += p_m1[:, dh:]
        out_ref[pl.ds(((my + 1) % N_DEV) * mc, mc), 0:dh] = commR[1, :, :].astype(
            jnp.float32
        )
        out_ref[pl.ds(((my - 1) % N_DEV) * mc, mc), dh:d] = commL[1, :, :].astype(
            jnp.float32
        )

        for h in range(3, 6):
            rR, rL = hop(h)
            rR.wait()
            rL.wait()
            r_slot = (h + 1) % 2
            cR = (my - (h - 3)) % N_DEV
            cL = (my + (h - 3)) % N_DEV
            out_ref[pl.ds(cR * mc, mc), 0:dh] = commR[r_slot, :, :].astype(
                jnp.float32
            )
            out_ref[pl.ds(cL * mc, mc), dh:d] = commL[r_slot, :, :].astype(
                jnp.float32
            )

    return pl.pallas_call(
        body,
        out_shape=jax.ShapeDtypeStruct((m, d), jnp.float32),
        in_specs=[pl.BlockSpec(memory_space=pltpu.VMEM)] * 4,
        out_specs=pl.BlockSpec(memory_space=pltpu.VMEM),
        scratch_shapes=[
            pltpu.VMEM((2, mc, dh), jnp.bfloat16),
            pltpu.VMEM((2, mc, dh), jnp.bfloat16),
            pltpu.SemaphoreType.DMA((2,)),
            pltpu.SemaphoreType.DMA((2,)),
            pltpu.SemaphoreType.DMA((2,)),
            pltpu.SemaphoreType.DMA((2,)),
        ],
        compiler_params=pltpu.CompilerParams(
            collective_id=0, vmem_limit_bytes=100 * 1024 * 1024
        ),
    )(x, Wg, Wu, Wd)
